# baseline (speedup 1.0000x reference)
import jax
from jax.experimental import pallas as pl

_ROWS = 1_000_000
_DIM = 16
_BLK = 118784
_GRID = (_ROWS + _BLK - 1) // _BLK


def _copy_body(u_ref, i_ref, ou_ref, oi_ref):
    ou_ref[...] = u_ref[...]
    oi_ref[...] = i_ref[...]


def kernel(user_table, item_table):
    spec = pl.BlockSpec((_DIM, _BLK), lambda k: (0, k))
    out = pl.pallas_call(
        _copy_body,
        grid=(_GRID,),
        in_specs=[spec, spec],
        out_specs=[spec, spec],
        out_shape=[
            jax.ShapeDtypeStruct((_DIM, _ROWS), user_table.dtype),
            jax.ShapeDtypeStruct((_DIM, _ROWS), item_table.dtype),
        ],
    )(user_table.T, item_table.T)
    return (out[0].T, out[1].T)


# speedup vs baseline: 1.0018x; 1.0018x over previous
import jax
from jax.experimental import pallas as pl

_ROWS = 1_000_000
_DIM = 16
_BLK = 119040
_GRID = (_ROWS + _BLK - 1) // _BLK


def _copy_body(u_ref, i_ref, ou_ref, oi_ref):
    ou_ref[...] = u_ref[...]
    oi_ref[...] = i_ref[...]


def kernel(user_table, item_table):
    spec = pl.BlockSpec((_DIM, _BLK), lambda k: (0, k))
    out = pl.pallas_call(
        _copy_body,
        grid=(_GRID,),
        in_specs=[spec, spec],
        out_specs=[spec, spec],
        out_shape=[
            jax.ShapeDtypeStruct((_DIM, _ROWS), user_table.dtype),
            jax.ShapeDtypeStruct((_DIM, _ROWS), item_table.dtype),
        ],
    )(user_table.T, item_table.T)
    return (out[0].T, out[1].T)
